# baseline (speedup 1.0000x reference)
import functools

import jax
import jax.numpy as jnp
from jax import lax
from jax.experimental import pallas as pl
from jax.experimental.pallas import tpu as pltpu
from jax.experimental.pallas import tpu_sc as plsc

B = 16384
NNZ = 26
D = 16
TOTAL = B * NNZ
NUM_WORKERS = 32
ROWS_PER_WORKER = TOTAL // NUM_WORKERS
CHUNK = 6656
NCHUNKS = ROWS_PER_WORKER // CHUNK
GROUPS = CHUNK // 16

_mesh = plsc.VectorSubcoreMesh(core_axis_name="c", subcore_axis_name="s")


@functools.partial(
    pl.kernel,
    mesh=_mesh,
    out_type=jax.ShapeDtypeStruct((TOTAL, D), jnp.float32),
    scratch_types=[
        pltpu.VMEM((CHUNK,), jnp.int32),
        pltpu.VMEM((GROUPS, 16), jnp.float32),
        pltpu.VMEM((CHUNK, D), jnp.float32),
        pltpu.SemaphoreType.DMA,
    ],
    compiler_params=pltpu.CompilerParams(use_tc_tiling_on_sc=False),
)
def _emb_sc(idx_hbm, val_hbm, table_hbm, out_hbm, idx_v, val_v, rows_v, sem):
    wid = lax.axis_index("s") * 2 + lax.axis_index("c")
    base = wid * ROWS_PER_WORKER

    def do_chunk(c, carry):
        cbase = pl.multiple_of(base + c * CHUNK, 512)
        pltpu.sync_copy(idx_hbm.at[pl.ds(cbase, CHUNK)], idx_v)
        pltpu.sync_copy(
            val_hbm.at[pl.ds(pl.multiple_of(cbase // 16, 32), GROUPS)], val_v
        )
        pltpu.async_copy(table_hbm.at[idx_v], rows_v, sem).wait()

        def scale_group(g, carry2):
            vals = val_v[g]
            r0 = g * 16
            for j in range(16):
                rows_v[r0 + j] = rows_v[r0 + j] * vals[j]
            return carry2

        lax.fori_loop(0, GROUPS, scale_group, 0)
        pltpu.sync_copy(rows_v, out_hbm.at[pl.ds(cbase, CHUNK)])
        return carry

    lax.fori_loop(0, NCHUNKS, do_chunk, 0)


def kernel(x, x_val, table):
    idx = x.reshape(TOTAL)
    val = x_val.reshape(TOTAL // 16, 16)
    out = _emb_sc(idx, val, table)
    return out.reshape(B, NNZ, D)
